# Initial kernel scaffold; baseline (speedup 1.0000x reference)
import jax
import jax.numpy as jnp
from jax.experimental import pallas as pl


def kernel(pointcloud, params):
    raise NotImplementedError("write your pallas kernel here")


# speedup vs baseline: 1.9762x; 1.9762x over previous
import functools

import jax
import jax.numpy as jnp
import numpy as np
from jax.experimental import pallas as pl
from jax.experimental.pallas import tpu as pltpu

_MLPS = [[32, 32], [64, 64], [128, 128], [256, 256]]
_FP_MLPS = [[64, 64], [128, 64], [256, 128], [512, 256]]
_CLS_FC = [64]
_NPOINTS = [4096, 1024, 256, 64]
_RADIUS = [0.1, 0.2, 0.4, 0.8]
_NSAMPLE = [32, 32, 32, 16]


def _lin_iota(shape, L):
    return (jax.lax.broadcasted_iota(jnp.int32, shape, 1) * L
            + jax.lax.broadcasted_iota(jnp.int32, shape, 2))


def _fps_level(x, y, z, o_ref, npoint):
    B, S, L = x.shape
    Lp = npoint // 8
    lin = _lin_iota((B, S, L), L)
    lin_p = _lin_iota((B, S, Lp), Lp)

    m0 = lin == 0
    cx0 = jnp.sum(jnp.sum(jnp.where(m0, x, 0.0), axis=2, keepdims=True),
                  axis=1, keepdims=True)
    cy0 = jnp.sum(jnp.sum(jnp.where(m0, y, 0.0), axis=2, keepdims=True),
                  axis=1, keepdims=True)
    cz0 = jnp.sum(jnp.sum(jnp.where(m0, z, 0.0), axis=2, keepdims=True),
                  axis=1, keepdims=True)

    def body(i, state):
        dist, cx, cy, cz = state
        m_out = lin_p == i
        o_ref[:, 0] = jnp.where(m_out, cx, o_ref[:, 0])
        o_ref[:, 1] = jnp.where(m_out, cy, o_ref[:, 1])
        o_ref[:, 2] = jnp.where(m_out, cz, o_ref[:, 2])
        dx = x - cx
        dy = y - cy
        dz = z - cz
        d = (dx * dx + dz * dz) + dy * dy
        dist = jnp.minimum(dist, d)
        mx = jnp.max(jnp.max(dist, axis=2, keepdims=True), axis=1,
                     keepdims=True)
        m = dist == mx
        far = jnp.min(jnp.min(jnp.where(m, lin, jnp.int32(2 ** 30)), axis=2,
                              keepdims=True), axis=1, keepdims=True)
        msel = lin == far
        ncx = jnp.sum(jnp.sum(jnp.where(msel, x, 0.0), axis=2, keepdims=True),
                      axis=1, keepdims=True)
        ncy = jnp.sum(jnp.sum(jnp.where(msel, y, 0.0), axis=2, keepdims=True),
                      axis=1, keepdims=True)
        ncz = jnp.sum(jnp.sum(jnp.where(msel, z, 0.0), axis=2, keepdims=True),
                      axis=1, keepdims=True)
        return dist, ncx, ncy, ncz

    dist0 = jnp.full((B, S, L), 1e10, jnp.float32)
    jax.lax.fori_loop(0, npoint, body, (dist0, cx0, cy0, cz0))


def _fps_kernel(x_ref, o1_ref, o2_ref, o3_ref, o4_ref):
    x = x_ref[:, 0]
    y = x_ref[:, 1]
    z = x_ref[:, 2]
    _fps_level(x, y, z, o1_ref, _NPOINTS[0])
    x = o1_ref[:, 0]
    y = o1_ref[:, 1]
    z = o1_ref[:, 2]
    _fps_level(x, y, z, o2_ref, _NPOINTS[1])
    x = o2_ref[:, 0]
    y = o2_ref[:, 1]
    z = o2_ref[:, 2]
    _fps_level(x, y, z, o3_ref, _NPOINTS[2])
    x = o3_ref[:, 0]
    y = o3_ref[:, 1]
    z = o3_ref[:, 2]
    _fps_level(x, y, z, o4_ref, _NPOINTS[3])


def _run_fps(xyz):
    B, N, _ = xyz.shape
    xt = xyz.transpose(0, 2, 1).reshape(B, 3, 8, N // 8)
    outs = pl.pallas_call(
        _fps_kernel,
        out_shape=tuple(
            jax.ShapeDtypeStruct((B, 3, 8, npt // 8), jnp.float32)
            for npt in _NPOINTS),
    )(xt)
    return [o.reshape(B, 3, npt).transpose(0, 2, 1)
            for o, npt in zip(outs, _NPOINTS)]


def _gather_pts(points, idx):
    return jax.vmap(lambda p, i: p[i])(points, idx)


def _pair_sqdist(a, b):
    return (jnp.sum(a * a, -1)[:, :, None] + jnp.sum(b * b, -1)[:, None, :]
            - 2.0 * jnp.einsum('bnc,bmc->bnm', a, b))


def _ball_group(radius, nsample, xyz, new_xyz):
    N = xyz.shape[1]
    d = _pair_sqdist(new_xyz, xyz)
    idx = jnp.where(d <= radius * radius,
                    jnp.arange(N, dtype=jnp.int32)[None, None, :], N)
    grp = -jax.lax.top_k(-idx, nsample)[0]
    first = grp[:, :, :1]
    grp = jnp.where(grp == N, jnp.broadcast_to(first, grp.shape), grp)
    grp = jnp.where(grp == N, 0, grp)
    return grp


def _mlp_chain(x, ps, axes):
    for W, b, g, be in ps:
        x = jnp.einsum('...c,cd->...d', x, W) + b
        m = jnp.mean(x, axis=axes, keepdims=True)
        v = jnp.var(x, axis=axes, keepdims=True)
        x = g * (x - m) * jax.lax.rsqrt(v + 1e-5) + be
        x = jax.nn.leaky_relu(x, negative_slope=0.2)
    return x


def _sa_stage(xyz, new_xyz, feats, radius, nsample, ps):
    gidx = _ball_group(radius, nsample, xyz, new_xyz)
    gxyz = _gather_pts(xyz, gidx) - new_xyz[:, :, None, :]
    if feats is not None:
        g = jnp.concatenate([gxyz, _gather_pts(feats, gidx)], -1)
    else:
        g = gxyz
    g = _mlp_chain(g, ps, (0, 1, 2))
    return jnp.max(g, axis=2)


def _fp_stage(xyz1, xyz2, f1, f2, ps):
    d = _pair_sqdist(xyz1, xyz2)
    nd, ni = jax.lax.top_k(-d, 3)
    nd = jnp.maximum(-nd, 1e-10)
    w = 1.0 / nd
    w = w / jnp.sum(w, -1, keepdims=True)
    interp = jnp.sum(_gather_pts(f2, ni) * w[..., None], axis=2)
    x = jnp.concatenate([f1, interp], -1) if f1 is not None else interp
    return _mlp_chain(x, ps, (0, 1))


def _fps_jnp(xyz, npoint):
    B, N, _ = xyz.shape

    def body(i, state):
        dist, far, idxs = state
        idxs = idxs.at[:, i].set(far)
        centroid = _gather_pts(xyz, far[:, None])
        d = jnp.sum((xyz - centroid) ** 2, -1)
        dist = jnp.minimum(dist, d)
        far = jnp.argmax(dist, -1).astype(jnp.int32)
        return (dist, far, idxs)

    init = (jnp.full((B, N), 1e10, jnp.float32), jnp.zeros((B,), jnp.int32),
            jnp.zeros((B, npoint), jnp.int32))
    _, _, idxs = jax.lax.fori_loop(0, npoint, body, init)
    return idxs


def _run_fps_jnp(xyz):
    outs = []
    cur = xyz
    for npt in _NPOINTS:
        fidx = _fps_jnp(cur, npt)
        cur = _gather_pts(cur, fidx)
        outs.append(cur)
    return outs


def kernel(pointcloud, params):
    xyz = pointcloud[..., :3]
    nxs = _run_fps(xyz)
    lx = [xyz] + nxs
    feats = None
    fs = []
    for i in range(4):
        feats = _sa_stage(lx[i], lx[i + 1], feats, _RADIUS[i], _NSAMPLE[i],
                          params['sa%d' % (i + 1)])
        fs.append(feats)
    l1, l2, l3, l4 = fs
    f3 = _fp_stage(lx[3], lx[4], l3, l4, params['fp4'])
    f2 = _fp_stage(lx[2], lx[3], l2, f3, params['fp3'])
    f1 = _fp_stage(lx[1], lx[2], l1, f2, params['fp2'])
    f0 = _fp_stage(lx[0], lx[1], None, f1, params['fp1'])
    h = _mlp_chain(f0, params['head'], (0, 1))
    return jnp.einsum('bnc,cd->bnd', h, params['head_W']) + params['head_b']


# speedup vs baseline: 3.2234x; 1.6311x over previous
import functools

import jax
import jax.numpy as jnp
import numpy as np
from jax import lax
from jax.experimental import pallas as pl
from jax.experimental.pallas import tpu as pltpu
from jax.experimental.pallas import tpu_sc as plsc

_MLPS = [[32, 32], [64, 64], [128, 128], [256, 256]]
_FP_MLPS = [[64, 64], [128, 64], [256, 128], [512, 256]]
_CLS_FC = [64]
_NPOINTS = [4096, 1024, 256, 64]
_RADIUS = [0.1, 0.2, 0.4, 0.8]
_NSAMPLE = [32, 32, 32, 16]


def _lin_iota(shape, L):
    return (jax.lax.broadcasted_iota(jnp.int32, shape, 1) * L
            + jax.lax.broadcasted_iota(jnp.int32, shape, 2))


def _fps_level(x, y, z, o_ref, npoint):
    B, S, L = x.shape
    Lp = npoint // 8
    lin = _lin_iota((B, S, L), L)
    lin_p = _lin_iota((B, S, Lp), Lp)

    m0 = lin == 0
    cx0 = jnp.sum(jnp.sum(jnp.where(m0, x, 0.0), axis=2, keepdims=True),
                  axis=1, keepdims=True)
    cy0 = jnp.sum(jnp.sum(jnp.where(m0, y, 0.0), axis=2, keepdims=True),
                  axis=1, keepdims=True)
    cz0 = jnp.sum(jnp.sum(jnp.where(m0, z, 0.0), axis=2, keepdims=True),
                  axis=1, keepdims=True)

    def body(i, state):
        dist, cx, cy, cz = state
        m_out = lin_p == i
        o_ref[:, 0] = jnp.where(m_out, cx, o_ref[:, 0])
        o_ref[:, 1] = jnp.where(m_out, cy, o_ref[:, 1])
        o_ref[:, 2] = jnp.where(m_out, cz, o_ref[:, 2])
        dx = x - cx
        dy = y - cy
        dz = z - cz
        d = (dx * dx + dz * dz) + dy * dy
        dist = jnp.minimum(dist, d)
        mx = jnp.max(jnp.max(dist, axis=2, keepdims=True), axis=1,
                     keepdims=True)
        m = dist == mx
        far = jnp.min(jnp.min(jnp.where(m, lin, jnp.int32(2 ** 30)), axis=2,
                              keepdims=True), axis=1, keepdims=True)
        msel = lin == far
        ncx = jnp.sum(jnp.sum(jnp.where(msel, x, 0.0), axis=2, keepdims=True),
                      axis=1, keepdims=True)
        ncy = jnp.sum(jnp.sum(jnp.where(msel, y, 0.0), axis=2, keepdims=True),
                      axis=1, keepdims=True)
        ncz = jnp.sum(jnp.sum(jnp.where(msel, z, 0.0), axis=2, keepdims=True),
                      axis=1, keepdims=True)
        return dist, ncx, ncy, ncz

    dist0 = jnp.full((B, S, L), 1e10, jnp.float32)
    jax.lax.fori_loop(0, npoint, body, (dist0, cx0, cy0, cz0))


def _fps_kernel(x_ref, o1_ref, o2_ref, o3_ref, o4_ref):
    x = x_ref[:, 0]
    y = x_ref[:, 1]
    z = x_ref[:, 2]
    _fps_level(x, y, z, o1_ref, _NPOINTS[0])
    x = o1_ref[:, 0]
    y = o1_ref[:, 1]
    z = o1_ref[:, 2]
    _fps_level(x, y, z, o2_ref, _NPOINTS[1])
    x = o2_ref[:, 0]
    y = o2_ref[:, 1]
    z = o2_ref[:, 2]
    _fps_level(x, y, z, o3_ref, _NPOINTS[2])
    x = o3_ref[:, 0]
    y = o3_ref[:, 1]
    z = o3_ref[:, 2]
    _fps_level(x, y, z, o4_ref, _NPOINTS[3])


def _run_fps(xyz):
    B, N, _ = xyz.shape
    xt = xyz.transpose(0, 2, 1).reshape(B, 3, 8, N // 8)
    outs = pl.pallas_call(
        _fps_kernel,
        out_shape=tuple(
            jax.ShapeDtypeStruct((B, 3, 8, npt // 8), jnp.float32)
            for npt in _NPOINTS),
    )(xt)
    return [o.reshape(B, 3, npt).transpose(0, 2, 1)
            for o, npt in zip(outs, _NPOINTS)]


def _gather_pts(points, idx):
    return jax.vmap(lambda p, i: p[i])(points, idx)


def _rup(x, m):
    return (x + m - 1) // m * m


def _make_ball_sc(N, M, r2, ns):
    Mt = M // 16
    CH = _rup(3 * Mt + 16, 128)
    OCH = _rup(Mt * ns + 16, 128)
    nchunk = N // 16
    mesh = plsc.VectorSubcoreMesh(core_axis_name="c", subcore_axis_name="s")

    BIG = jnp.int32(2 ** 30)

    def body(dmat_hbm, out_hbm, row, obuf):
        b = lax.axis_index("c")
        t = lax.axis_index("s")
        lane = lax.iota(jnp.int32, 16)
        row_base = ((b * 16 + t) * Mt) * N

        def per_m(m, carry):
            pltpu.sync_copy(dmat_hbm.at[pl.ds(row_base + m * N, N)], row)
            base = m * ns

            def per_chunk(j, state):
                cnt, first = state
                d = row[pl.ds(j * 16, 16)]
                msk = d <= r2
                idxs = lane + j * 16
                start = base + jnp.minimum(cnt, ns)
                plsc.store_compressed(obuf.at[pl.ds(start, 16)], idxs,
                                      mask=msk & (cnt < ns))
                nm = jnp.sum(jnp.where(msk, 1, 0).astype(jnp.int32))
                cand = jnp.min(jnp.where(msk, idxs, BIG))
                return cnt + nm, jnp.minimum(first, cand)

            cnt, first = lax.fori_loop(
                0, nchunk, per_chunk, (jnp.int32(0), BIG), unroll=4)
            for k in range(ns // 16):
                sl = pl.ds(base + k * 16, 16)
                slot = lane + k * 16
                obuf[sl] = jnp.where(slot < cnt, obuf[sl], first)
            return carry

        lax.fori_loop(0, Mt, per_m, 0)
        pltpu.sync_copy(obuf, out_hbm.at[pl.ds((b * 16 + t) * OCH, OCH)])

    return pl.kernel(
        body,
        out_type=jax.ShapeDtypeStruct((32 * OCH,), jnp.int32),
        mesh=mesh,
        scratch_types=[
            pltpu.VMEM((N,), jnp.float32),
            pltpu.VMEM((OCH,), jnp.int32),
        ],
        compiler_params=pltpu.CompilerParams(needs_layout_passes=False),
    )


def _ball_group_sc(level, xyz, new_xyz):
    B, N, _ = xyz.shape
    M = _NPOINTS[level]
    ns = _NSAMPLE[level]
    r = _RADIUS[level]
    r2 = np.float32(r * r)
    Mt = M // 16
    OCH = _rup(Mt * ns + 16, 128)
    dmat = _pair_sqdist(new_xyz, xyz).reshape(-1)
    fn = _make_ball_sc(N, M, float(r2), ns)
    out = fn(dmat).reshape(B, 16, OCH)[:, :, :Mt * ns]
    return out.reshape(B, M, ns)


def _pair_sqdist(a, b):
    return (jnp.sum(a * a, -1)[:, :, None] + jnp.sum(b * b, -1)[:, None, :]
            - 2.0 * jnp.einsum('bnc,bmc->bnm', a, b))


def _ball_group(radius, nsample, xyz, new_xyz):
    N = xyz.shape[1]
    d = _pair_sqdist(new_xyz, xyz)
    idx = jnp.where(d <= radius * radius,
                    jnp.arange(N, dtype=jnp.int32)[None, None, :], N)
    grp = -jax.lax.top_k(-idx, nsample)[0]
    first = grp[:, :, :1]
    grp = jnp.where(grp == N, jnp.broadcast_to(first, grp.shape), grp)
    grp = jnp.where(grp == N, 0, grp)
    return grp


def _mlp_chain(x, ps, axes):
    for W, b, g, be in ps:
        x = jnp.einsum('...c,cd->...d', x, W) + b
        m = jnp.mean(x, axis=axes, keepdims=True)
        v = jnp.var(x, axis=axes, keepdims=True)
        x = g * (x - m) * jax.lax.rsqrt(v + 1e-5) + be
        x = jax.nn.leaky_relu(x, negative_slope=0.2)
    return x


def _sa_stage(level, xyz, new_xyz, feats, radius, nsample, ps):
    gidx = _ball_group_sc(level, xyz, new_xyz)
    gxyz = _gather_pts(xyz, gidx) - new_xyz[:, :, None, :]
    if feats is not None:
        g = jnp.concatenate([gxyz, _gather_pts(feats, gidx)], -1)
    else:
        g = gxyz
    g = _mlp_chain(g, ps, (0, 1, 2))
    return jnp.max(g, axis=2)


def _fp_stage(xyz1, xyz2, f1, f2, ps):
    d = _pair_sqdist(xyz1, xyz2)
    nd, ni = jax.lax.top_k(-d, 3)
    nd = jnp.maximum(-nd, 1e-10)
    w = 1.0 / nd
    w = w / jnp.sum(w, -1, keepdims=True)
    interp = jnp.sum(_gather_pts(f2, ni) * w[..., None], axis=2)
    x = jnp.concatenate([f1, interp], -1) if f1 is not None else interp
    return _mlp_chain(x, ps, (0, 1))


def _fps_jnp(xyz, npoint):
    B, N, _ = xyz.shape

    def body(i, state):
        dist, far, idxs = state
        idxs = idxs.at[:, i].set(far)
        centroid = _gather_pts(xyz, far[:, None])
        d = jnp.sum((xyz - centroid) ** 2, -1)
        dist = jnp.minimum(dist, d)
        far = jnp.argmax(dist, -1).astype(jnp.int32)
        return (dist, far, idxs)

    init = (jnp.full((B, N), 1e10, jnp.float32), jnp.zeros((B,), jnp.int32),
            jnp.zeros((B, npoint), jnp.int32))
    _, _, idxs = jax.lax.fori_loop(0, npoint, body, init)
    return idxs


def _run_fps_jnp(xyz):
    outs = []
    cur = xyz
    for npt in _NPOINTS:
        fidx = _fps_jnp(cur, npt)
        cur = _gather_pts(cur, fidx)
        outs.append(cur)
    return outs


def kernel(pointcloud, params):
    xyz = pointcloud[..., :3]
    nxs = _run_fps(xyz)
    lx = [xyz] + nxs
    feats = None
    fs = []
    for i in range(4):
        feats = _sa_stage(i, lx[i], lx[i + 1], feats, _RADIUS[i], _NSAMPLE[i],
                          params['sa%d' % (i + 1)])
        fs.append(feats)
    l1, l2, l3, l4 = fs
    f3 = _fp_stage(lx[3], lx[4], l3, l4, params['fp4'])
    f2 = _fp_stage(lx[2], lx[3], l2, f3, params['fp3'])
    f1 = _fp_stage(lx[1], lx[2], l1, f2, params['fp2'])
    f0 = _fp_stage(lx[0], lx[1], None, f1, params['fp1'])
    h = _mlp_chain(f0, params['head'], (0, 1))
    return jnp.einsum('bnc,cd->bnd', h, params['head_W']) + params['head_b']


# speedup vs baseline: 7.0723x; 2.1940x over previous
import functools

import jax
import jax.numpy as jnp
import numpy as np
from jax import lax
from jax.experimental import pallas as pl
from jax.experimental.pallas import tpu as pltpu
from jax.experimental.pallas import tpu_sc as plsc

_MLPS = [[32, 32], [64, 64], [128, 128], [256, 256]]
_FP_MLPS = [[64, 64], [128, 64], [256, 128], [512, 256]]
_CLS_FC = [64]
_NPOINTS = [4096, 1024, 256, 64]
_RADIUS = [0.1, 0.2, 0.4, 0.8]
_NSAMPLE = [32, 32, 32, 16]


def _lin_iota(shape, L):
    return (jax.lax.broadcasted_iota(jnp.int32, shape, 1) * L
            + jax.lax.broadcasted_iota(jnp.int32, shape, 2))


def _fps_level(x, y, z, o_ref, npoint):
    B, S, L = x.shape
    Lp = npoint // 8
    lin = _lin_iota((B, S, L), L)
    lin_p = _lin_iota((B, S, Lp), Lp)

    m0 = lin == 0
    cx0 = jnp.sum(jnp.sum(jnp.where(m0, x, 0.0), axis=2, keepdims=True),
                  axis=1, keepdims=True)
    cy0 = jnp.sum(jnp.sum(jnp.where(m0, y, 0.0), axis=2, keepdims=True),
                  axis=1, keepdims=True)
    cz0 = jnp.sum(jnp.sum(jnp.where(m0, z, 0.0), axis=2, keepdims=True),
                  axis=1, keepdims=True)

    def body(i, state):
        dist, cx, cy, cz = state
        m_out = lin_p == i
        o_ref[:, 0] = jnp.where(m_out, cx, o_ref[:, 0])
        o_ref[:, 1] = jnp.where(m_out, cy, o_ref[:, 1])
        o_ref[:, 2] = jnp.where(m_out, cz, o_ref[:, 2])
        dx = x - cx
        dy = y - cy
        dz = z - cz
        d = (dx * dx + dz * dz) + dy * dy
        dist = jnp.minimum(dist, d)
        mx = jnp.max(jnp.max(dist, axis=2, keepdims=True), axis=1,
                     keepdims=True)
        m = dist == mx
        far = jnp.min(jnp.min(jnp.where(m, lin, jnp.int32(2 ** 30)), axis=2,
                              keepdims=True), axis=1, keepdims=True)
        msel = lin == far
        ncx = jnp.sum(jnp.sum(jnp.where(msel, x, 0.0), axis=2, keepdims=True),
                      axis=1, keepdims=True)
        ncy = jnp.sum(jnp.sum(jnp.where(msel, y, 0.0), axis=2, keepdims=True),
                      axis=1, keepdims=True)
        ncz = jnp.sum(jnp.sum(jnp.where(msel, z, 0.0), axis=2, keepdims=True),
                      axis=1, keepdims=True)
        return dist, ncx, ncy, ncz

    dist0 = jnp.full((B, S, L), 1e10, jnp.float32)
    jax.lax.fori_loop(0, npoint, body, (dist0, cx0, cy0, cz0))


def _fps_kernel(x_ref, o1_ref, o2_ref, o3_ref, o4_ref):
    x = x_ref[:, 0]
    y = x_ref[:, 1]
    z = x_ref[:, 2]
    _fps_level(x, y, z, o1_ref, _NPOINTS[0])
    x = o1_ref[:, 0]
    y = o1_ref[:, 1]
    z = o1_ref[:, 2]
    _fps_level(x, y, z, o2_ref, _NPOINTS[1])
    x = o2_ref[:, 0]
    y = o2_ref[:, 1]
    z = o2_ref[:, 2]
    _fps_level(x, y, z, o3_ref, _NPOINTS[2])
    x = o3_ref[:, 0]
    y = o3_ref[:, 1]
    z = o3_ref[:, 2]
    _fps_level(x, y, z, o4_ref, _NPOINTS[3])


def _run_fps(xyz):
    B, N, _ = xyz.shape
    xt = xyz.transpose(0, 2, 1).reshape(B, 3, 8, N // 8)
    outs = pl.pallas_call(
        _fps_kernel,
        out_shape=tuple(
            jax.ShapeDtypeStruct((B, 3, 8, npt // 8), jnp.float32)
            for npt in _NPOINTS),
    )(xt)
    return [o.reshape(B, 3, npt).transpose(0, 2, 1)
            for o, npt in zip(outs, _NPOINTS)]


def _gather_pts(points, idx):
    return jax.vmap(lambda p, i: p[i])(points, idx)


def _rup(x, m):
    return (x + m - 1) // m * m


def _make_ball_sc(N, M, r2, ns):
    Mt = M // 16
    CH = _rup(3 * Mt + 16, 128)
    OCH = _rup(Mt * ns + 16, 128)
    nchunk = N // 16
    mesh = plsc.VectorSubcoreMesh(core_axis_name="c", subcore_axis_name="s")

    BIG = jnp.int32(2 ** 30)

    def body(dmat_hbm, out_hbm, row, obuf):
        b = lax.axis_index("c")
        t = lax.axis_index("s")
        lane = lax.iota(jnp.int32, 16)
        row_base = ((b * 16 + t) * Mt) * N

        def per_m(m, carry):
            pltpu.sync_copy(dmat_hbm.at[pl.ds(row_base + m * N, N)], row)
            base = m * ns

            def per_chunk(j, state):
                cnt, first = state
                d = row[pl.ds(j * 16, 16)]
                msk = d <= r2
                idxs = lane + j * 16
                start = base + jnp.minimum(cnt, ns)
                plsc.store_compressed(obuf.at[pl.ds(start, 16)], idxs,
                                      mask=msk & (cnt < ns))
                nm = jnp.sum(jnp.where(msk, 1, 0).astype(jnp.int32))
                cand = jnp.min(jnp.where(msk, idxs, BIG))
                return cnt + nm, jnp.minimum(first, cand)

            cnt, first = lax.fori_loop(
                0, nchunk, per_chunk, (jnp.int32(0), BIG), unroll=4)
            for k in range(ns // 16):
                sl = pl.ds(base + k * 16, 16)
                slot = lane + k * 16
                obuf[sl] = jnp.where(slot < cnt, obuf[sl], first)
            return carry

        lax.fori_loop(0, Mt, per_m, 0)
        pltpu.sync_copy(obuf, out_hbm.at[pl.ds((b * 16 + t) * OCH, OCH)])

    return pl.kernel(
        body,
        out_type=jax.ShapeDtypeStruct((32 * OCH,), jnp.int32),
        mesh=mesh,
        scratch_types=[
            pltpu.VMEM((N,), jnp.float32),
            pltpu.VMEM((OCH,), jnp.int32),
        ],
        compiler_params=pltpu.CompilerParams(needs_layout_passes=False),
    )


def _ball_group_sc(level, xyz, new_xyz):
    B, N, _ = xyz.shape
    M = _NPOINTS[level]
    ns = _NSAMPLE[level]
    r = _RADIUS[level]
    r2 = np.float32(r * r)
    Mt = M // 16
    OCH = _rup(Mt * ns + 16, 128)
    dmat = _pair_sqdist(new_xyz, xyz).reshape(-1)
    fn = _make_ball_sc(N, M, float(r2), ns)
    out = fn(dmat).reshape(B, 16, OCH)[:, :, :Mt * ns]
    return out.reshape(B, M, ns)


def _pair_sqdist(a, b):
    return (jnp.sum(a * a, -1)[:, :, None] + jnp.sum(b * b, -1)[:, None, :]
            - 2.0 * jnp.einsum('bnc,bmc->bnm', a, b))


def _ball_group(radius, nsample, xyz, new_xyz):
    N = xyz.shape[1]
    d = _pair_sqdist(new_xyz, xyz)
    idx = jnp.where(d <= radius * radius,
                    jnp.arange(N, dtype=jnp.int32)[None, None, :], N)
    grp = -jax.lax.top_k(-idx, nsample)[0]
    first = grp[:, :, :1]
    grp = jnp.where(grp == N, jnp.broadcast_to(first, grp.shape), grp)
    grp = jnp.where(grp == N, 0, grp)
    return grp


def _mlp_chain(x, ps, axes):
    for W, b, g, be in ps:
        x = jnp.einsum('...c,cd->...d', x, W) + b
        m = jnp.mean(x, axis=axes, keepdims=True)
        v = jnp.var(x, axis=axes, keepdims=True)
        x = g * (x - m) * jax.lax.rsqrt(v + 1e-5) + be
        x = jax.nn.leaky_relu(x, negative_slope=0.2)
    return x


def _sa_stage(level, xyz, new_xyz, feats, radius, nsample, ps):
    gidx = _ball_group_sc(level, xyz, new_xyz)
    gxyz = _gather_pts(xyz, gidx) - new_xyz[:, :, None, :]
    if feats is not None:
        g = jnp.concatenate([gxyz, _gather_pts(feats, gidx)], -1)
    else:
        g = gxyz
    g = _mlp_chain(g, ps, (0, 1, 2))
    return jnp.max(g, axis=2)


def _make_three_nn_sc(n1, n2):
    Qt = n1 // 16
    N2P = _rup(n2, 128)
    OI = _rup(Qt * 3 + 32, 128)
    nchunk = N2P // 16
    mesh = plsc.VectorSubcoreMesh(core_axis_name="c", subcore_axis_name="s")
    BIG = jnp.int32(2 ** 30)
    INF = jnp.float32(np.inf)

    def body(dmat_hbm, oi_hbm, od_hbm, row, obi, obd):
        b = lax.axis_index("c")
        t = lax.axis_index("s")
        lane = lax.iota(jnp.int32, 16)

        def scan_min(i1, i2):
            def per_chunk(j, state):
                bv, bi = state
                d = row[pl.ds(j * 16, 16)]
                idx = lane + j * 16
                ok = (d < bv) & (idx != i1) & (idx != i2)
                return jnp.where(ok, d, bv), jnp.where(ok, idx, bi)

            bv, bi = lax.fori_loop(
                0, nchunk, per_chunk,
                (jnp.full((16,), INF), jnp.full((16,), BIG)), unroll=4)
            mv = jnp.min(bv)
            mi = jnp.min(jnp.where(bv == mv, bi, BIG))
            return mv, mi

        def per_q(q, carry):
            roff = ((b * 16 + t) * Qt + q) * N2P
            pltpu.sync_copy(dmat_hbm.at[pl.ds(roff, N2P)], row)
            v1, i1 = scan_min(-1, -1)
            v2, i2 = scan_min(i1, -1)
            v3, i3 = scan_min(i1, i2)
            vi = jnp.where(lane == 0, i1, jnp.where(lane == 1, i2,
                           jnp.where(lane == 2, i3, 0)))
            vd = jnp.where(lane == 0, v1, jnp.where(lane == 1, v2,
                           jnp.where(lane == 2, v3, 0.0)))
            obi[pl.ds(q * 3, 16)] = vi
            obd[pl.ds(q * 3, 16)] = vd
            return carry

        lax.fori_loop(0, Qt, per_q, 0)
        pltpu.sync_copy(obi, oi_hbm.at[pl.ds((b * 16 + t) * OI, OI)])
        pltpu.sync_copy(obd, od_hbm.at[pl.ds((b * 16 + t) * OI, OI)])

    return pl.kernel(
        body,
        out_type=(jax.ShapeDtypeStruct((32 * OI,), jnp.int32),
                  jax.ShapeDtypeStruct((32 * OI,), jnp.float32)),
        mesh=mesh,
        scratch_types=[
            pltpu.VMEM((N2P,), jnp.float32),
            pltpu.VMEM((OI,), jnp.int32),
            pltpu.VMEM((OI,), jnp.float32),
        ],
        compiler_params=pltpu.CompilerParams(needs_layout_passes=False),
    )


def _three_nn_sc(xyz1, xyz2):
    B, n1, _ = xyz1.shape
    n2 = xyz2.shape[1]
    Qt = n1 // 16
    N2P = _rup(n2, 128)
    OI = _rup(Qt * 3 + 32, 128)
    d = _pair_sqdist(xyz1, xyz2)
    if N2P != n2:
        d = jnp.pad(d, ((0, 0), (0, 0), (0, N2P - n2)),
                    constant_values=np.inf)
    fn = _make_three_nn_sc(n1, n2)
    oi, od = fn(d.reshape(-1))
    ni = oi.reshape(B, 16, OI)[:, :, :Qt * 3].reshape(B, n1, 3)
    nd = od.reshape(B, 16, OI)[:, :, :Qt * 3].reshape(B, n1, 3)
    return ni, nd


def _fp_stage(xyz1, xyz2, f1, f2, ps):
    ni, ndraw = _three_nn_sc(xyz1, xyz2)
    nd = jnp.maximum(ndraw, 1e-10)
    w = 1.0 / nd
    w = w / jnp.sum(w, -1, keepdims=True)
    interp = jnp.sum(_gather_pts(f2, ni) * w[..., None], axis=2)
    x = jnp.concatenate([f1, interp], -1) if f1 is not None else interp
    return _mlp_chain(x, ps, (0, 1))


def _fps_jnp(xyz, npoint):
    B, N, _ = xyz.shape

    def body(i, state):
        dist, far, idxs = state
        idxs = idxs.at[:, i].set(far)
        centroid = _gather_pts(xyz, far[:, None])
        d = jnp.sum((xyz - centroid) ** 2, -1)
        dist = jnp.minimum(dist, d)
        far = jnp.argmax(dist, -1).astype(jnp.int32)
        return (dist, far, idxs)

    init = (jnp.full((B, N), 1e10, jnp.float32), jnp.zeros((B,), jnp.int32),
            jnp.zeros((B, npoint), jnp.int32))
    _, _, idxs = jax.lax.fori_loop(0, npoint, body, init)
    return idxs


def _run_fps_jnp(xyz):
    outs = []
    cur = xyz
    for npt in _NPOINTS:
        fidx = _fps_jnp(cur, npt)
        cur = _gather_pts(cur, fidx)
        outs.append(cur)
    return outs


def kernel(pointcloud, params):
    xyz = pointcloud[..., :3]
    nxs = _run_fps(xyz)
    lx = [xyz] + nxs
    feats = None
    fs = []
    for i in range(4):
        feats = _sa_stage(i, lx[i], lx[i + 1], feats, _RADIUS[i], _NSAMPLE[i],
                          params['sa%d' % (i + 1)])
        fs.append(feats)
    l1, l2, l3, l4 = fs
    f3 = _fp_stage(lx[3], lx[4], l3, l4, params['fp4'])
    f2 = _fp_stage(lx[2], lx[3], l2, f3, params['fp3'])
    f1 = _fp_stage(lx[1], lx[2], l1, f2, params['fp2'])
    f0 = _fp_stage(lx[0], lx[1], None, f1, params['fp1'])
    h = _mlp_chain(f0, params['head'], (0, 1))
    return jnp.einsum('bnc,cd->bnd', h, params['head_W']) + params['head_b']
